# baseline (speedup 1.0000x reference)
import functools

import jax
import jax.numpy as jnp
from jax import lax
from jax.experimental import pallas as pl
from jax.experimental.pallas import tpu as pltpu
from jax.experimental.pallas import tpu_sc as plsc

_D = 2048
_K = 307
_NC = 2
_NS = 16
_NW = _NC * _NS
_CH = 16
_CHD = _CH * _D
_NB = 256
_U = 8


def _sc_gating_mask(xf):
    R = xf.shape[0] // _D
    rows_per_w = R // _NW
    nch = rows_per_w // _CH

    mesh = plsc.VectorSubcoreMesh(core_axis_name="c", subcore_axis_name="s")

    @functools.partial(
        pl.kernel,
        mesh=mesh,
        compiler_params=pltpu.CompilerParams(needs_layout_passes=False),
        out_type=jax.ShapeDtypeStruct((R * _D,), jnp.float32),
        scratch_types=[
            pltpu.VMEM((_CHD,), jnp.float32),
            pltpu.VMEM((_CHD,), jnp.float32),
            pltpu.VMEM((_CHD,), jnp.float32),
            pltpu.VMEM((_NB * 16,), jnp.int32),
            pltpu.SemaphoreType.DMA,
            pltpu.SemaphoreType.DMA,
            pltpu.SemaphoreType.DMA,
        ],
    )
    def k(x_hbm, m_hbm, in0, in1, outb, hist, s_in0, s_in1, s_out):
        wid = lax.axis_index("c") * _NS + lax.axis_index("s")
        base = wid * rows_per_w * _D
        lane = lax.iota(jnp.int32, 16)
        ones = jnp.full((16,), 1, jnp.int32)
        zeros = jnp.zeros((16,), jnp.int32)
        kvec = jnp.full((16,), _K, jnp.int32)
        cbj = [lane * _D + j for j in range(_U)]

        def off(i):
            return base + i * _CHD

        def start_in(i, buf, sem):
            pltpu.make_async_copy(
                x_hbm.at[pl.ds(off(i), _CHD)], buf, sem
            ).start()

        def wait_in(i, buf, sem):
            pltpu.make_async_copy(
                x_hbm.at[pl.ds(off(i), _CHD)], buf, sem
            ).wait()

        def clr(j, c):
            hist[pl.ds(j * 16, 16)] = zeros
            return c

        lax.fori_loop(0, _NB, clr, 0)

        def hist_pass(buf, addr_fn, match_fn):
            def body(db, c):
                d0 = db * _U
                for j in range(_U):
                    v = plsc.load_gather(buf, [cbj[j] + d0])
                    u = lax.bitcast_convert_type(v, jnp.int32) & jnp.int32(
                        0x7FFFFFFF
                    )
                    addr = addr_fn(u) | lane
                    if match_fn is None:
                        plsc.addupdate_scatter(hist, [addr], ones)
                    else:
                        plsc.addupdate_scatter(
                            hist, [addr], ones, mask=match_fn(u)
                        )
                return c

            lax.fori_loop(0, _D // _U, body, 0)

        def scan(nb, kk):
            def body(jb, carry):
                acc, nc_cnt, cnt_above = carry
                for j in range(4):
                    beta = nb - 1 - (jb * 4 + j)
                    v = hist[pl.ds(beta * 16, 16)]
                    hist[pl.ds(beta * 16, 16)] = zeros
                    acc = acc + v
                    nc = acc < kk
                    nc_cnt = nc_cnt + jnp.where(nc, 1, 0)
                    cnt_above = jnp.where(nc, acc, cnt_above)
                return acc, nc_cnt, cnt_above

            _, nc_cnt, cnt_above = lax.fori_loop(
                0, nb // 4, body, (zeros, zeros, zeros)
            )
            return (nb - 1) - nc_cnt, kk - cnt_above

        def process(i, buf, sem):
            wait_in(i, buf, sem)

            hist_pass(buf, lambda u: (u >> 19) & jnp.int32(0xFF0), None)
            b1, kk = scan(_NB, kvec)
            pfx = b1
            hist_pass(
                buf,
                lambda u: (u >> 11) & jnp.int32(0xFF0),
                lambda u: (u >> 23) == pfx,
            )
            b2, kk = scan(_NB, kk)
            pfx = (pfx << 8) | b2
            hist_pass(
                buf,
                lambda u: (u >> 3) & jnp.int32(0xFF0),
                lambda u: (u >> 15) == pfx,
            )
            b3, kk = scan(_NB, kk)
            pfx = (pfx << 8) | b3
            hist_pass(
                buf,
                lambda u: (u << 4) & jnp.int32(0x7F0),
                lambda u: (u >> 7) == pfx,
            )
            b4, kk = scan(128, kk)
            tvec = (pfx << 7) | b4

            @pl.when(i > 0)
            def _():
                pltpu.make_async_copy(
                    outb, m_hbm.at[pl.ds(off(i) - _CHD, _CHD)], s_out
                ).wait()

            def mbody(db, c):
                d0 = db * _U
                for j in range(_U):
                    idx = cbj[j] + d0
                    v = plsc.load_gather(buf, [idx])
                    u = lax.bitcast_convert_type(v, jnp.int32) & jnp.int32(
                        0x7FFFFFFF
                    )
                    m = jnp.where(u >= tvec, 1.0, 0.0).astype(jnp.float32)
                    plsc.store_scatter(outb, [idx], m)
                return c

            lax.fori_loop(0, _D // _U, mbody, 0)

            pltpu.make_async_copy(
                outb, m_hbm.at[pl.ds(off(i), _CHD)], s_out
            ).start()

            @pl.when(i + 2 < nch)
            def _():
                start_in(i + 2, buf, sem)

        start_in(0, in0, s_in0)
        start_in(1, in1, s_in1)

        def pair(p, c):
            process(p * 2, in0, s_in0)
            process(p * 2 + 1, in1, s_in1)
            return c

        lax.fori_loop(0, nch // 2, pair, 0)

        pltpu.make_async_copy(
            outb, m_hbm.at[pl.ds(off(nch - 1), _CHD)], s_out
        ).wait()

    return k(xf)


def kernel(x):
    B, T, D = x.shape
    xf = x.reshape(B * T * D)
    mask = _sc_gating_mask(xf)
    return x, mask.reshape(B, T, D)


# speedup vs baseline: 1.7716x; 1.7716x over previous
import functools

import jax
import jax.numpy as jnp
from jax import lax
from jax.experimental import pallas as pl
from jax.experimental.pallas import tpu as pltpu
from jax.experimental.pallas import tpu_sc as plsc

_D = 2048
_K = 307
_NC = 2
_NS = 16
_NW = _NC * _NS
_CH = 16
_CHD = _CH * _D
_PS = 2056
_NB = 256
_U = 8


def _sc_gating_mask(xf):
    R = xf.shape[0] // _D
    rows_per_w = R // _NW
    nch = rows_per_w // _CH

    mesh = plsc.VectorSubcoreMesh(core_axis_name="c", subcore_axis_name="s")

    @functools.partial(
        pl.kernel,
        mesh=mesh,
        compiler_params=pltpu.CompilerParams(needs_layout_passes=False),
        out_type=jax.ShapeDtypeStruct((R * _D,), jnp.float32),
        scratch_types=[
            pltpu.VMEM((_CH * _PS,), jnp.float32),
            pltpu.VMEM((_CH * _PS,), jnp.float32),
            pltpu.VMEM((_CH * _PS,), jnp.float32),
            pltpu.VMEM((_NB * 16,), jnp.int32),
            pltpu.SemaphoreType.DMA,
            pltpu.SemaphoreType.DMA,
            pltpu.SemaphoreType.DMA,
        ],
    )
    def k(x_hbm, m_hbm, in0, in1, outb, hist, s_in0, s_in1, s_out):
        wid = lax.axis_index("c") * _NS + lax.axis_index("s")
        base = wid * rows_per_w * _D
        lane = lax.iota(jnp.int32, 16)
        ones = jnp.full((16,), 1, jnp.int32)
        zeros = jnp.zeros((16,), jnp.int32)
        kvec = jnp.full((16,), _K, jnp.int32)
        cbj = [lane * _PS + j for j in range(_U)]

        def off(i):
            return base + i * _CHD

        def start_in(i, buf, sem):
            for r in range(_CH):
                pltpu.make_async_copy(
                    x_hbm.at[pl.ds(off(i) + r * _D, _D)],
                    buf.at[pl.ds(r * _PS, _D)],
                    sem,
                ).start()

        def wait_in(i, buf, sem):
            pltpu.make_async_copy(
                x_hbm.at[pl.ds(off(i), _CHD)],
                buf.at[pl.ds(0, _CHD)],
                sem,
            ).wait()

        def clr(j, c):
            hist[pl.ds(j * 16, 16)] = zeros
            return c

        lax.fori_loop(0, _NB, clr, 0)

        def hist_pass(buf, addr_fn, match_fn):
            def body(db, c):
                d0 = db * _U
                for j in range(_U):
                    v = plsc.load_gather(buf, [cbj[j] + d0])
                    u = lax.bitcast_convert_type(v, jnp.int32) & jnp.int32(
                        0x7FFFFFFF
                    )
                    addr = addr_fn(u) | lane
                    if match_fn is None:
                        plsc.addupdate_scatter(hist, [addr], ones)
                    else:
                        plsc.addupdate_scatter(
                            hist, [addr], ones, mask=match_fn(u)
                        )
                return c

            lax.fori_loop(0, _D // _U, body, 0)

        def scan(nb, kk):
            def body(jb, carry):
                acc, nc_cnt, cnt_above = carry
                for j in range(4):
                    beta = nb - 1 - (jb * 4 + j)
                    v = hist[pl.ds(beta * 16, 16)]
                    hist[pl.ds(beta * 16, 16)] = zeros
                    acc = acc + v
                    nc = acc < kk
                    nc_cnt = nc_cnt + jnp.where(nc, 1, 0)
                    cnt_above = jnp.where(nc, acc, cnt_above)
                return acc, nc_cnt, cnt_above

            _, nc_cnt, cnt_above = lax.fori_loop(
                0, nb // 4, body, (zeros, zeros, zeros)
            )
            return (nb - 1) - nc_cnt, kk - cnt_above

        def process(i, buf, sem):
            wait_in(i, buf, sem)

            hist_pass(buf, lambda u: (u >> 19) & jnp.int32(0xFF0), None)
            b1, kk = scan(_NB, kvec)
            pfx = b1
            hist_pass(
                buf,
                lambda u: (u >> 11) & jnp.int32(0xFF0),
                lambda u: (u >> 23) == pfx,
            )
            b2, kk = scan(_NB, kk)
            pfx = (pfx << 8) | b2
            hist_pass(
                buf,
                lambda u: (u >> 3) & jnp.int32(0xFF0),
                lambda u: (u >> 15) == pfx,
            )
            b3, kk = scan(_NB, kk)
            pfx = (pfx << 8) | b3
            hist_pass(
                buf,
                lambda u: (u << 4) & jnp.int32(0x7F0),
                lambda u: (u >> 7) == pfx,
            )
            b4, kk = scan(128, kk)
            tvec = (pfx << 7) | b4

            @pl.when(i > 0)
            def _():
                pltpu.make_async_copy(
                    outb.at[pl.ds(0, _CHD)],
                    m_hbm.at[pl.ds(off(i) - _CHD, _CHD)],
                    s_out,
                ).wait()

            def mbody(db, c):
                d0 = db * _U
                for j in range(_U):
                    idx = cbj[j] + d0
                    v = plsc.load_gather(buf, [idx])
                    u = lax.bitcast_convert_type(v, jnp.int32) & jnp.int32(
                        0x7FFFFFFF
                    )
                    m = jnp.where(u >= tvec, 1.0, 0.0).astype(jnp.float32)
                    plsc.store_scatter(outb, [idx], m)
                return c

            lax.fori_loop(0, _D // _U, mbody, 0)

            for r in range(_CH):
                pltpu.make_async_copy(
                    outb.at[pl.ds(r * _PS, _D)],
                    m_hbm.at[pl.ds(off(i) + r * _D, _D)],
                    s_out,
                ).start()

            @pl.when(i + 2 < nch)
            def _():
                start_in(i + 2, buf, sem)

        start_in(0, in0, s_in0)
        start_in(1, in1, s_in1)

        def pair(p, c):
            process(p * 2, in0, s_in0)
            process(p * 2 + 1, in1, s_in1)
            return c

        lax.fori_loop(0, nch // 2, pair, 0)

        pltpu.make_async_copy(
            outb.at[pl.ds(0, _CHD)],
            m_hbm.at[pl.ds(off(nch - 1), _CHD)],
            s_out,
        ).wait()

    return k(xf)


def kernel(x):
    B, T, D = x.shape
    xf = x.reshape(B * T * D)
    mask = _sc_gating_mask(xf)
    return x, mask.reshape(B, T, D)


# speedup vs baseline: 2.0909x; 1.1802x over previous
import functools

import jax
import jax.numpy as jnp
from jax import lax
from jax.experimental import pallas as pl
from jax.experimental.pallas import tpu as pltpu
from jax.experimental.pallas import tpu_sc as plsc

_D = 2048
_K = 307
_NC = 2
_NS = 16
_NW = _NC * _NS
_CH = 16
_CHD = _CH * _D

_NB = 256
_U = 8


def _sc_gating_mask(xf):
    R = xf.shape[0] // _D
    rows_per_w = R // _NW
    nch = rows_per_w // _CH

    mesh = plsc.VectorSubcoreMesh(core_axis_name="c", subcore_axis_name="s")

    @functools.partial(
        pl.kernel,
        mesh=mesh,
        compiler_params=pltpu.CompilerParams(needs_layout_passes=False),
        out_type=jax.ShapeDtypeStruct((R * _D,), jnp.float32),
        scratch_types=[
            pltpu.VMEM((_CHD,), jnp.float32),
            pltpu.VMEM((_CHD,), jnp.float32),
            pltpu.VMEM((_CHD,), jnp.float32),
            pltpu.VMEM((_NB * 16,), jnp.int32),
            pltpu.SemaphoreType.DMA,
            pltpu.SemaphoreType.DMA,
            pltpu.SemaphoreType.DMA,
        ],
    )
    def k(x_hbm, m_hbm, in0, in1, outb, hist, s_in0, s_in1, s_out):
        wid = lax.axis_index("c") * _NS + lax.axis_index("s")
        base = wid * rows_per_w * _D
        lane = lax.iota(jnp.int32, 16)
        ones = jnp.full((16,), 1, jnp.int32)
        zeros = jnp.zeros((16,), jnp.int32)
        kvec = jnp.full((16,), _K, jnp.int32)
        cbj = [lane * _D + (lane ^ j) for j in range(_U)]

        def off(i):
            return base + i * _CHD

        def start_in(i, buf, sem):
            pltpu.make_async_copy(
                x_hbm.at[pl.ds(off(i), _CHD)], buf, sem
            ).start()

        def wait_in(i, buf, sem):
            pltpu.make_async_copy(
                x_hbm.at[pl.ds(off(i), _CHD)], buf, sem
            ).wait()

        def clr(j, c):
            hist[pl.ds(j * 16, 16)] = zeros
            return c

        lax.fori_loop(0, _NB, clr, 0)

        def hist_pass(buf, addr_fn, match_fn):
            def body(db, c):
                d0 = db * _U
                for j in range(_U):
                    v = plsc.load_gather(buf, [cbj[j] ^ d0])
                    u = lax.bitcast_convert_type(v, jnp.int32) & jnp.int32(
                        0x7FFFFFFF
                    )
                    addr = addr_fn(u) | lane
                    if match_fn is None:
                        plsc.addupdate_scatter(hist, [addr], ones)
                    else:
                        plsc.addupdate_scatter(
                            hist, [addr], ones, mask=match_fn(u)
                        )
                return c

            lax.fori_loop(0, _D // _U, body, 0)

        def scan(nb, kk):
            def body(jb, carry):
                acc, nc_cnt, cnt_above = carry
                for j in range(4):
                    beta = nb - 1 - (jb * 4 + j)
                    v = hist[pl.ds(beta * 16, 16)]
                    hist[pl.ds(beta * 16, 16)] = zeros
                    acc = acc + v
                    nc = acc < kk
                    nc_cnt = nc_cnt + jnp.where(nc, 1, 0)
                    cnt_above = jnp.where(nc, acc, cnt_above)
                return acc, nc_cnt, cnt_above

            _, nc_cnt, cnt_above = lax.fori_loop(
                0, nb // 4, body, (zeros, zeros, zeros)
            )
            return (nb - 1) - nc_cnt, kk - cnt_above

        def process(i, buf, sem):
            wait_in(i, buf, sem)

            hist_pass(buf, lambda u: (u >> 19) & jnp.int32(0xFF0), None)
            b1, kk = scan(_NB, kvec)
            pfx = b1
            hist_pass(
                buf,
                lambda u: (u >> 11) & jnp.int32(0xFF0),
                lambda u: (u >> 23) == pfx,
            )
            b2, kk = scan(_NB, kk)
            pfx = (pfx << 8) | b2
            hist_pass(
                buf,
                lambda u: (u >> 3) & jnp.int32(0xFF0),
                lambda u: (u >> 15) == pfx,
            )
            b3, kk = scan(_NB, kk)
            pfx = (pfx << 8) | b3
            hist_pass(
                buf,
                lambda u: (u << 4) & jnp.int32(0x7F0),
                lambda u: (u >> 7) == pfx,
            )
            b4, kk = scan(128, kk)
            tvec = (pfx << 7) | b4

            @pl.when(i > 0)
            def _():
                pltpu.make_async_copy(
                    outb, m_hbm.at[pl.ds(off(i) - _CHD, _CHD)], s_out
                ).wait()

            def mbody(db, c):
                d0 = db * _U
                for j in range(_U):
                    idx = cbj[j] ^ d0
                    v = plsc.load_gather(buf, [idx])
                    u = lax.bitcast_convert_type(v, jnp.int32) & jnp.int32(
                        0x7FFFFFFF
                    )
                    m = jnp.where(u >= tvec, 1.0, 0.0).astype(jnp.float32)
                    plsc.store_scatter(outb, [idx], m)
                return c

            lax.fori_loop(0, _D // _U, mbody, 0)

            pltpu.make_async_copy(
                outb, m_hbm.at[pl.ds(off(i), _CHD)], s_out
            ).start()

            @pl.when(i + 2 < nch)
            def _():
                start_in(i + 2, buf, sem)

        start_in(0, in0, s_in0)
        start_in(1, in1, s_in1)

        def pair(p, c):
            process(p * 2, in0, s_in0)
            process(p * 2 + 1, in1, s_in1)
            return c

        lax.fori_loop(0, nch // 2, pair, 0)

        pltpu.make_async_copy(
            outb, m_hbm.at[pl.ds(off(nch - 1), _CHD)], s_out
        ).wait()

    return k(xf)


def kernel(x):
    B, T, D = x.shape
    xf = x.reshape(B * T * D)
    mask = _sc_gating_mask(xf)
    return x, mask.reshape(B, T, D)


# speedup vs baseline: 7.2102x; 3.4484x over previous
import functools

import jax
import jax.numpy as jnp
from jax import lax
from jax.experimental import pallas as pl
from jax.experimental.pallas import tpu as pltpu
from jax.experimental.pallas import tpu_sc as plsc

_D = 2048
_K = 307
_NC = 2
_NS = 16
_NW = _NC * _NS
_CH = 16
_CHD = _CH * _D
_NB = 256


def _sc_gating_mask(xf):
    R = xf.shape[0] // _D
    rows_per_w = R // _NW
    nch = rows_per_w // _CH

    mesh = plsc.VectorSubcoreMesh(core_axis_name="c", subcore_axis_name="s")

    @functools.partial(
        pl.kernel,
        mesh=mesh,
        compiler_params=pltpu.CompilerParams(needs_layout_passes=False),
        out_type=jax.ShapeDtypeStruct((R * _D,), jnp.float32),
        scratch_types=[
            pltpu.VMEM((_CHD,), jnp.float32),
            pltpu.VMEM((_CHD,), jnp.int32),
            pltpu.VMEM((_CHD,), jnp.float32),
            pltpu.VMEM((_NB * 16,), jnp.int32),
            pltpu.SemaphoreType.DMA,
            pltpu.SemaphoreType.DMA,
        ],
    )
    def k(x_hbm, m_hbm, inb, ubuf, outb, hist, s_in, s_out):
        wid = lax.axis_index("c") * _NS + lax.axis_index("s")
        base = wid * rows_per_w * _D
        lane = lax.iota(jnp.int32, 16)
        ones = jnp.full((16,), 1, jnp.int32)
        zeros = jnp.zeros((16,), jnp.int32)
        kvec = jnp.full((16,), _K, jnp.int32)
        rowbase = lane * _D

        def off(i):
            return base + i * _CHD

        def start_in(i):
            pltpu.make_async_copy(
                x_hbm.at[pl.ds(off(i), _CHD)], inb, s_in
            ).start()

        @plsc.parallel_loop(0, _NB, unroll=4)
        def _(j):
            hist[pl.ds(j * 16, 16)] = zeros

        def scan(nb, kk):
            @plsc.parallel_loop(0, nb, unroll=4, carry=(zeros, zeros, zeros))
            def res(j, carry):
                acc, nc_cnt, cnt_above = carry
                beta = nb - 1 - j
                v = hist[pl.ds(beta * 16, 16)]
                hist[pl.ds(beta * 16, 16)] = zeros
                acc = acc + v
                nc = acc < kk
                nc_cnt = nc_cnt + jnp.where(nc, 1, 0)
                cnt_above = jnp.where(nc, acc, cnt_above)
                return acc, nc_cnt, cnt_above

            _, nc_cnt, cnt_above = res
            return (nb - 1) - nc_cnt, kk - cnt_above

        def refine_pass(shift_cmp, pfx, shift_addr):
            @plsc.parallel_loop(0, _D, unroll=8)
            def _(d):
                u = ubuf[pl.ds(d * 16, 16)]
                m = (u >> shift_cmp) == pfx
                addr = ((u >> shift_addr) & jnp.int32(0xFF0)) | lane
                plsc.addupdate_scatter(hist, [addr], ones, mask=m)

        def process(i, c):
            pltpu.make_async_copy(
                x_hbm.at[pl.ds(off(i), _CHD)], inb, s_in
            ).wait()

            @plsc.parallel_loop(0, _D, unroll=8)
            def _(d):
                idx = rowbase | (d ^ lane)
                v = plsc.load_gather(inb, [idx])
                u = lax.bitcast_convert_type(v, jnp.int32) & jnp.int32(
                    0x7FFFFFFF
                )
                plsc.store_scatter(ubuf, [(d << 4) | lane], u)
                addr = ((u >> 19) & jnp.int32(0xFF0)) | lane
                plsc.addupdate_scatter(hist, [addr], ones)

            b1, kk = scan(_NB, kvec)
            pfx = b1

            @pl.when(i + 1 < nch)
            def _():
                start_in(i + 1)

            refine_pass(23, pfx, 11)
            b2, kk = scan(_NB, kk)
            pfx = (pfx << 8) | b2
            refine_pass(15, pfx, 3)
            b3, kk = scan(_NB, kk)
            pfx = (pfx << 8) | b3

            @plsc.parallel_loop(0, _D, unroll=8)
            def _(d):
                u = ubuf[pl.ds(d * 16, 16)]
                m = (u >> 7) == pfx
                addr = ((u << 4) & jnp.int32(0x7F0)) | lane
                plsc.addupdate_scatter(hist, [addr], ones, mask=m)

            b4, kk = scan(128, kk)
            tvec = (pfx << 7) | b4

            @pl.when(i > 0)
            def _():
                pltpu.make_async_copy(
                    outb, m_hbm.at[pl.ds(off(i) - _CHD, _CHD)], s_out
                ).wait()

            @plsc.parallel_loop(0, _D, unroll=8)
            def _(d):
                u = ubuf[pl.ds(d * 16, 16)]
                m = jnp.where(u >= tvec, 1.0, 0.0).astype(jnp.float32)
                plsc.store_scatter(outb, [rowbase | (d ^ lane)], m)

            pltpu.make_async_copy(
                outb, m_hbm.at[pl.ds(off(i), _CHD)], s_out
            ).start()
            return c

        start_in(0)
        lax.fori_loop(0, nch, process, 0)

        pltpu.make_async_copy(
            outb, m_hbm.at[pl.ds(off(nch - 1), _CHD)], s_out
        ).wait()

    return k(xf)


def kernel(x):
    B, T, D = x.shape
    xf = x.reshape(B * T * D)
    mask = _sc_gating_mask(xf)
    return x, mask.reshape(B, T, D)


# speedup vs baseline: 7.3137x; 1.0144x over previous
import functools

import jax
import jax.numpy as jnp
from jax import lax
from jax.experimental import pallas as pl
from jax.experimental.pallas import tpu as pltpu
from jax.experimental.pallas import tpu_sc as plsc

_D = 2048
_K = 307
_NC = 2
_NS = 16
_NW = _NC * _NS
_CH = 16
_CHD = _CH * _D
_NB = 256


def _sc_gating_mask(xf):
    R = xf.shape[0] // _D
    rows_per_w = R // _NW
    nch = rows_per_w // _CH

    mesh = plsc.VectorSubcoreMesh(core_axis_name="c", subcore_axis_name="s")

    @functools.partial(
        pl.kernel,
        mesh=mesh,
        compiler_params=pltpu.CompilerParams(needs_layout_passes=False),
        out_type=jax.ShapeDtypeStruct((R * _D,), jnp.float32),
        scratch_types=[
            pltpu.VMEM((_CHD,), jnp.float32),
            pltpu.VMEM((_CHD,), jnp.int32),
            pltpu.VMEM((_CHD,), jnp.float32),
            pltpu.VMEM((_NB * 16,), jnp.int32),
            pltpu.SemaphoreType.DMA,
            pltpu.SemaphoreType.DMA,
        ],
    )
    def k(x_hbm, m_hbm, inb, ubuf, outb, hist, s_in, s_out):
        wid = lax.axis_index("c") * _NS + lax.axis_index("s")
        base = wid * rows_per_w * _D
        lane = lax.iota(jnp.int32, 16)
        ones = jnp.full((16,), 1, jnp.int32)
        zeros = jnp.zeros((16,), jnp.int32)
        kvec = jnp.full((16,), _K, jnp.int32)
        rowbase = lane * _D

        def off(i):
            return base + i * _CHD

        def start_in(i):
            pltpu.make_async_copy(
                x_hbm.at[pl.ds(off(i), _CHD)], inb, s_in
            ).start()

        @plsc.parallel_loop(0, _NB, unroll=4)
        def _(j):
            hist[pl.ds(j * 16, 16)] = zeros

        def scan(nb, kk):
            @plsc.parallel_loop(0, nb, unroll=4, carry=(zeros, zeros, zeros))
            def res(j, carry):
                acc, nc_cnt, cnt_above = carry
                beta = nb - 1 - j
                v = hist[pl.ds(beta * 16, 16)]
                hist[pl.ds(beta * 16, 16)] = zeros
                acc = acc + v
                nc = acc < kk
                nc_cnt = nc_cnt + jnp.where(nc, 1, 0)
                cnt_above = jnp.where(nc, acc, cnt_above)
                return acc, nc_cnt, cnt_above

            _, nc_cnt, cnt_above = res
            return (nb - 1) - nc_cnt, kk - cnt_above

        def refine_pass(shift_cmp, pfx, shift_addr):
            @plsc.parallel_loop(0, _D, unroll=16)
            def _(d):
                u = ubuf[pl.ds(d * 16, 16)]
                m = (u >> shift_cmp) == pfx
                addr = ((u >> shift_addr) & jnp.int32(0xFF0)) | lane
                plsc.addupdate_scatter(hist, [addr], ones, mask=m)

        def process(i, c):
            pltpu.make_async_copy(
                x_hbm.at[pl.ds(off(i), _CHD)], inb, s_in
            ).wait()

            @plsc.parallel_loop(0, _D, unroll=16)
            def _(d):
                idx = rowbase | (d ^ lane)
                v = plsc.load_gather(inb, [idx])
                u = lax.bitcast_convert_type(v, jnp.int32) & jnp.int32(
                    0x7FFFFFFF
                )
                plsc.store_scatter(ubuf, [(d << 4) | lane], u)
                addr = ((u >> 19) & jnp.int32(0xFF0)) | lane
                plsc.addupdate_scatter(hist, [addr], ones)

            b1, kk = scan(_NB, kvec)
            pfx = b1

            @pl.when(i + 1 < nch)
            def _():
                start_in(i + 1)

            refine_pass(23, pfx, 11)
            b2, kk = scan(_NB, kk)
            pfx = (pfx << 8) | b2
            refine_pass(15, pfx, 3)
            b3, kk = scan(_NB, kk)
            pfx = (pfx << 8) | b3

            @plsc.parallel_loop(0, _D, unroll=16)
            def _(d):
                u = ubuf[pl.ds(d * 16, 16)]
                m = (u >> 7) == pfx
                addr = ((u << 4) & jnp.int32(0x7F0)) | lane
                plsc.addupdate_scatter(hist, [addr], ones, mask=m)

            b4, kk = scan(128, kk)
            tvec = (pfx << 7) | b4

            @pl.when(i > 0)
            def _():
                pltpu.make_async_copy(
                    outb, m_hbm.at[pl.ds(off(i) - _CHD, _CHD)], s_out
                ).wait()

            @plsc.parallel_loop(0, _D, unroll=16)
            def _(d):
                u = ubuf[pl.ds(d * 16, 16)]
                m = jnp.where(u >= tvec, 1.0, 0.0).astype(jnp.float32)
                plsc.store_scatter(outb, [rowbase | (d ^ lane)], m)

            pltpu.make_async_copy(
                outb, m_hbm.at[pl.ds(off(i), _CHD)], s_out
            ).start()
            return c

        start_in(0)
        lax.fori_loop(0, nch, process, 0)

        pltpu.make_async_copy(
            outb, m_hbm.at[pl.ds(off(nch - 1), _CHD)], s_out
        ).wait()

    return k(xf)


def kernel(x):
    B, T, D = x.shape
    xf = x.reshape(B * T * D)
    mask = _sc_gating_mask(xf)
    return x, mask.reshape(B, T, D)


# speedup vs baseline: 9.1690x; 1.2537x over previous
import functools

import jax
import jax.numpy as jnp
from jax import lax
from jax.experimental import pallas as pl
from jax.experimental.pallas import tpu as pltpu
from jax.experimental.pallas import tpu_sc as plsc

_D = 2048
_K = 307
_NC = 2
_NS = 16
_NW = _NC * _NS
_CH = 16
_CHD = _CH * _D
_NB = 256


def _sc_gating_mask(xf):
    R = xf.shape[0]
    rows_per_w = R // _NW
    nch = rows_per_w // _CH

    mesh = plsc.VectorSubcoreMesh(core_axis_name="c", subcore_axis_name="s")

    @functools.partial(
        pl.kernel,
        mesh=mesh,
        compiler_params=pltpu.CompilerParams(
            needs_layout_passes=False, use_tc_tiling_on_sc=True
        ),
        out_type=jax.ShapeDtypeStruct((R, _D), jnp.float32),
        scratch_types=[
            pltpu.VMEM((_CH, _D), jnp.float32),
            pltpu.VMEM((_CHD,), jnp.int32),
            pltpu.VMEM((_CH, _D), jnp.float32),
            pltpu.VMEM((_NB * 16,), jnp.int32),
            pltpu.SemaphoreType.DMA,
            pltpu.SemaphoreType.DMA,
        ],
    )
    def k(x_hbm, m_hbm, inb, ubuf, outb, hist, s_in, s_out):
        wid = lax.axis_index("c") * _NS + lax.axis_index("s")
        rbase = wid * rows_per_w
        lane = lax.iota(jnp.int32, 16)
        ones = jnp.full((16,), 1, jnp.int32)
        zeros = jnp.zeros((16,), jnp.int32)
        kvec = jnp.full((16,), _K, jnp.int32)

        def roff(i):
            return rbase + i * _CH

        def start_in(i):
            pltpu.make_async_copy(
                x_hbm.at[pl.ds(roff(i), _CH), :], inb, s_in
            ).start()

        @plsc.parallel_loop(0, _NB, unroll=4)
        def _(j):
            hist[pl.ds(j * 16, 16)] = zeros

        def scan(nb, kk):
            @plsc.parallel_loop(0, nb, unroll=4, carry=(zeros, zeros, zeros))
            def res(j, carry):
                acc, nc_cnt, cnt_above = carry
                beta = nb - 1 - j
                v = hist[pl.ds(beta * 16, 16)]
                hist[pl.ds(beta * 16, 16)] = zeros
                acc = acc + v
                nc = acc < kk
                nc_cnt = nc_cnt + jnp.where(nc, 1, 0)
                cnt_above = jnp.where(nc, acc, cnt_above)
                return acc, nc_cnt, cnt_above

            _, nc_cnt, cnt_above = res
            return (nb - 1) - nc_cnt, kk - cnt_above

        def refine_pass(shift_cmp, pfx, shift_addr):
            @plsc.parallel_loop(0, _D, unroll=16)
            def _(d):
                u = ubuf[pl.ds(d * 16, 16)]
                m = (u >> shift_cmp) == pfx
                addr = ((u >> shift_addr) & jnp.int32(0xFF0)) | lane
                plsc.addupdate_scatter(hist, [addr], ones, mask=m)

        def process(i, c):
            pltpu.make_async_copy(
                x_hbm.at[pl.ds(roff(i), _CH), :], inb, s_in
            ).wait()

            @plsc.parallel_loop(0, _D, unroll=16)
            def _(d):
                v = plsc.load_gather(inb, [lane, d ^ lane])
                u = lax.bitcast_convert_type(v, jnp.int32) & jnp.int32(
                    0x7FFFFFFF
                )
                plsc.store_scatter(ubuf, [(d << 4) | lane], u)
                addr = ((u >> 19) & jnp.int32(0xFF0)) | lane
                plsc.addupdate_scatter(hist, [addr], ones)

            b1, kk = scan(_NB, kvec)
            pfx = b1

            @pl.when(i + 1 < nch)
            def _():
                start_in(i + 1)

            refine_pass(23, pfx, 11)
            b2, kk = scan(_NB, kk)
            pfx = (pfx << 8) | b2
            refine_pass(15, pfx, 3)
            b3, kk = scan(_NB, kk)
            pfx = (pfx << 8) | b3

            @plsc.parallel_loop(0, _D, unroll=16)
            def _(d):
                u = ubuf[pl.ds(d * 16, 16)]
                m = (u >> 7) == pfx
                addr = ((u << 4) & jnp.int32(0x7F0)) | lane
                plsc.addupdate_scatter(hist, [addr], ones, mask=m)

            b4, kk = scan(128, kk)
            tvec = (pfx << 7) | b4

            @pl.when(i > 0)
            def _():
                pltpu.make_async_copy(
                    outb, m_hbm.at[pl.ds(roff(i) - _CH, _CH), :], s_out
                ).wait()

            @plsc.parallel_loop(0, _D, unroll=16)
            def _(d):
                u = ubuf[pl.ds(d * 16, 16)]
                m = jnp.where(u >= tvec, 1.0, 0.0).astype(jnp.float32)
                plsc.store_scatter(outb, [lane, d ^ lane], m)

            pltpu.make_async_copy(
                outb, m_hbm.at[pl.ds(roff(i), _CH), :], s_out
            ).start()
            return c

        start_in(0)
        lax.fori_loop(0, nch, process, 0)

        pltpu.make_async_copy(
            outb, m_hbm.at[pl.ds(roff(nch - 1), _CH), :], s_out
        ).wait()

    return k(xf)


def kernel(x):
    B, T, D = x.shape
    xf = x.reshape(B * T, D)
    mask = _sc_gating_mask(xf)
    return x, mask.reshape(B, T, D)


# speedup vs baseline: 9.5602x; 1.0427x over previous
import functools

import jax
import jax.numpy as jnp
from jax import lax
from jax.experimental import pallas as pl
from jax.experimental.pallas import tpu as pltpu
from jax.experimental.pallas import tpu_sc as plsc

_D = 2048
_K = 307
_NC = 2
_NS = 16
_NW = _NC * _NS
_CH = 16
_NB1 = 1024
_NBS = 128
_CAP = 1008


def _sc_gating_mask(xf):
    R = xf.shape[0]
    rows_per_w = R // _NW
    nch = rows_per_w // _CH

    mesh = plsc.VectorSubcoreMesh(core_axis_name="c", subcore_axis_name="s")

    @functools.partial(
        pl.kernel,
        mesh=mesh,
        compiler_params=pltpu.CompilerParams(
            needs_layout_passes=False, use_tc_tiling_on_sc=True
        ),
        out_type=jax.ShapeDtypeStruct((R, _D), jnp.float32),
        scratch_types=[
            pltpu.VMEM((_CH, _D), jnp.float32),
            pltpu.VMEM((_CH, _D), jnp.float32),
            pltpu.VMEM((_CH, _D), jnp.float32),
            pltpu.VMEM((_NB1 * 16,), jnp.int32),
            pltpu.VMEM((_CAP * 16,), jnp.int32),
            pltpu.VMEM((16,), jnp.int32),
            pltpu.SemaphoreType.DMA,
            pltpu.SemaphoreType.DMA,
            pltpu.SemaphoreType.DMA,
        ],
    )
    def k(x_hbm, m_hbm, in0, in1, outb, hist, cand, tref, s0, s1, s_out):
        wid = lax.axis_index("c") * _NS + lax.axis_index("s")
        rbase = wid * rows_per_w
        lane = lax.iota(jnp.int32, 16)
        ones = jnp.full((16,), 1, jnp.int32)
        zeros = jnp.zeros((16,), jnp.int32)
        kvec = jnp.full((16,), _K, jnp.int32)

        def roff(i):
            return rbase + i * _CH

        def start_in(i, buf, sem):
            pltpu.make_async_copy(
                x_hbm.at[pl.ds(roff(i), _CH), :], buf, sem
            ).start()

        def wait_in(i, buf, sem):
            pltpu.make_async_copy(
                x_hbm.at[pl.ds(roff(i), _CH), :], buf, sem
            ).wait()

        @plsc.parallel_loop(0, _NB1, unroll=4)
        def _(j):
            hist[pl.ds(j * 16, 16)] = zeros

        def scan(nb, kk):
            @plsc.parallel_loop(0, nb, unroll=4, carry=(zeros, zeros, zeros))
            def res(j, carry):
                acc, nc_cnt, cnt_above = carry
                beta = nb - 1 - j
                v = hist[pl.ds(beta * 16, 16)]
                hist[pl.ds(beta * 16, 16)] = zeros
                acc = acc + v
                nc = acc < kk
                nc_cnt = nc_cnt + jnp.where(nc, 1, 0)
                cnt_above = jnp.where(nc, acc, cnt_above)
                return acc, nc_cnt, cnt_above

            _, nc_cnt, cnt_above = res
            return (nb - 1) - nc_cnt, kk - cnt_above

        def gat_u(buf, d):
            v = plsc.load_gather(buf, [lane, d ^ lane])
            return lax.bitcast_convert_type(v, jnp.int32) & jnp.int32(
                0x7FFFFFFF
            )

        def refine(kk1, b1, loop_hi, load_u, valid_fn):
            @plsc.parallel_loop(0, loop_hi, unroll=4)
            def _(j):
                u = load_u(j)
                m = valid_fn(j, u >> 21, b1)
                addr = ((u >> 10) & jnp.int32(0x7F0)) | lane
                plsc.addupdate_scatter(hist, [addr], ones, mask=m)

            b2, kk2 = scan(_NBS, kk1)
            pfx = (b1 << 7) | b2

            @plsc.parallel_loop(0, loop_hi, unroll=4)
            def _(j):
                u = load_u(j)
                m = valid_fn(j, u >> 14, pfx)
                addr = ((u >> 3) & jnp.int32(0x7F0)) | lane
                plsc.addupdate_scatter(hist, [addr], ones, mask=m)

            b3, kk3 = scan(_NBS, kk2)
            pfx = (pfx << 7) | b3

            @plsc.parallel_loop(0, loop_hi, unroll=4)
            def _(j):
                u = load_u(j)
                m = valid_fn(j, u >> 7, pfx)
                addr = ((u << 4) & jnp.int32(0x7F0)) | lane
                plsc.addupdate_scatter(hist, [addr], ones, mask=m)

            b4, _ = scan(_NBS, kk3)
            tref[...] = (pfx << 7) | b4

        def process(i, buf, sem):
            wait_in(i, buf, sem)

            @plsc.parallel_loop(0, _D, unroll=8)
            def _(d):
                u = gat_u(buf, d)
                addr = ((u >> 17) & jnp.int32(0x3FF0)) | lane
                plsc.addupdate_scatter(hist, [addr], ones)

            b1, kk1 = scan(_NB1, kvec)

            @plsc.parallel_loop(0, _D, unroll=8, carry=zeros)
            def cnt(d, c):
                u = gat_u(buf, d)
                m = ((u >> 21) == b1) & (c < _CAP)
                plsc.store_scatter(cand, [(c << 4) | lane], u, mask=m)
                return c + jnp.where(m, 1, 0)

            m_max = jnp.max(cnt)

            refine(
                kk1,
                b1,
                m_max,
                lambda j: cand[pl.ds(j * 16, 16)],
                lambda j, upart, pfx: (j < cnt) & (upart == pfx),
            )

            @pl.when(m_max >= _CAP)
            def _():
                refine(
                    kk1,
                    b1,
                    _D,
                    lambda d: gat_u(buf, d),
                    lambda d, upart, pfx: upart == pfx,
                )

            tvec = tref[...]

            @pl.when(i > 0)
            def _():
                pltpu.make_async_copy(
                    outb, m_hbm.at[pl.ds(roff(i) - _CH, _CH), :], s_out
                ).wait()

            @plsc.parallel_loop(0, _D, unroll=8)
            def _(d):
                u = gat_u(buf, d)
                m = jnp.where(u >= tvec, 1.0, 0.0).astype(jnp.float32)
                plsc.store_scatter(outb, [lane, d ^ lane], m)

            pltpu.make_async_copy(
                outb, m_hbm.at[pl.ds(roff(i), _CH), :], s_out
            ).start()

            @pl.when(i + 2 < nch)
            def _():
                start_in(i + 2, buf, sem)

        start_in(0, in0, s0)
        start_in(1, in1, s1)

        def pair(p, c):
            process(p * 2, in0, s0)
            process(p * 2 + 1, in1, s1)
            return c

        lax.fori_loop(0, nch // 2, pair, 0)

        pltpu.make_async_copy(
            outb, m_hbm.at[pl.ds(roff(nch - 1), _CH), :], s_out
        ).wait()

    return k(xf)


def kernel(x):
    B, T, D = x.shape
    xf = x.reshape(B * T, D)
    mask = _sc_gating_mask(xf)
    return x, mask.reshape(B, T, D)
